# baseline (speedup 1.0000x reference)
import jax
import jax.numpy as jnp
from jax import lax
from jax.experimental import pallas as pl
from jax.experimental.pallas import tpu as pltpu
from jax.experimental.pallas import tpu_sc as plsc

N_NODES = 100000
NUM_SPECIES = 100
NUM_WORKERS = 32
LANES = 16
CHUNK = 3136


def _gather_body(idx_hbm, e0_hbm, out_hbm, tbl_v, idx_v, out_v, sem0, sem1):
    wid = lax.axis_index("s") * 2 + lax.axis_index("c")
    base = lax.min(wid * CHUNK, N_NODES - CHUNK)
    pltpu.sync_copy(idx_hbm.at[pl.ds(base, CHUNK)], idx_v)
    pltpu.async_copy(e0_hbm.at[idx_v], out_v, sem0).wait()
    pltpu.sync_copy(out_v, out_hbm.at[pl.ds(base, CHUNK)])


def kernel(node_species, E0):
    mesh = plsc.VectorSubcoreMesh(core_axis_name="c", subcore_axis_name="s")
    run = pl.kernel(
        _gather_body,
        out_type=jax.ShapeDtypeStruct((N_NODES,), jnp.float32),
        mesh=mesh,
        scratch_types=[
            pltpu.VMEM((NUM_SPECIES,), jnp.float32),
            pltpu.VMEM((CHUNK,), jnp.int32),
            pltpu.VMEM((CHUNK,), jnp.float32),
            pltpu.SemaphoreType.DMA,
            pltpu.SemaphoreType.DMA,
        ],
        compiler_params=pltpu.CompilerParams(
            needs_layout_passes=False,
            disable_bounds_checks=True,
            disable_semaphore_checks=True,
            skip_device_barrier=True,
        ),
    )
    return run(node_species.astype(jnp.int32), E0.astype(jnp.float32))


# speedup vs baseline: 23.4524x; 23.4524x over previous
import jax
import jax.numpy as jnp
from jax import lax
from jax.experimental import pallas as pl
from jax.experimental.pallas import tpu as pltpu
from jax.experimental.pallas import tpu_sc as plsc

N_NODES = 100000
NUM_SPECIES = 100
NUM_WORKERS = 32
LANES = 16
CHUNK = 3136


def _gather_body(idx_hbm, e0_hbm, out_hbm, tbl_v, idx_v, out_v, sem0, sem1):
    wid = lax.axis_index("s") * 2 + lax.axis_index("c")
    base = lax.min(wid * CHUNK, N_NODES - CHUNK)
    c_tbl = pltpu.async_copy(e0_hbm, tbl_v, sem0)
    c_idx = pltpu.async_copy(idx_hbm.at[pl.ds(base, CHUNK)], idx_v, sem1)
    c_tbl.wait()
    c_idx.wait()

    @plsc.parallel_loop(0, CHUNK // LANES, unroll=7)
    def _(j):
        off = j * LANES
        iv = idx_v[pl.ds(off, LANES)]
        out_v[pl.ds(off, LANES)] = plsc.load_gather(tbl_v, [iv])

    pltpu.sync_copy(out_v, out_hbm.at[pl.ds(base, CHUNK)])


def kernel(node_species, E0):
    mesh = plsc.VectorSubcoreMesh(core_axis_name="c", subcore_axis_name="s")
    run = pl.kernel(
        _gather_body,
        out_type=jax.ShapeDtypeStruct((N_NODES,), jnp.float32),
        mesh=mesh,
        scratch_types=[
            pltpu.VMEM((NUM_SPECIES,), jnp.float32),
            pltpu.VMEM((CHUNK,), jnp.int32),
            pltpu.VMEM((CHUNK,), jnp.float32),
            pltpu.SemaphoreType.DMA,
            pltpu.SemaphoreType.DMA,
        ],
        compiler_params=pltpu.CompilerParams(
            needs_layout_passes=False,
            disable_bounds_checks=True,
            disable_semaphore_checks=True,
            skip_device_barrier=True,
        ),
    )
    return run(node_species.astype(jnp.int32), E0.astype(jnp.float32))


# speedup vs baseline: 23.5288x; 1.0033x over previous
import jax
import jax.numpy as jnp
from jax import lax
from jax.experimental import pallas as pl
from jax.experimental.pallas import tpu as pltpu
from jax.experimental.pallas import tpu_sc as plsc

N_NODES = 100000
NUM_SPECIES = 100
NUM_WORKERS = 32
LANES = 16
CHUNK = 3136


def _gather_body(idx_hbm, e0_hbm, out_hbm, tbl_v, idx_v, out_v, sem0, sem1):
    wid = lax.axis_index("s") * 2 + lax.axis_index("c")
    base = lax.min(wid * CHUNK, N_NODES - CHUNK)
    c_tbl = pltpu.async_copy(e0_hbm, tbl_v, sem0)
    c_idx = pltpu.async_copy(idx_hbm.at[pl.ds(base, CHUNK)], idx_v, sem1)
    c_tbl.wait()
    c_idx.wait()

    @plsc.parallel_loop(0, CHUNK // LANES, unroll=7)
    def _(j):
        off = j * LANES
        iv = idx_v[pl.ds(off, LANES)]
        out_v[pl.ds(off, LANES)] = plsc.load_gather(tbl_v, [iv])

    pltpu.sync_copy(out_v, out_hbm.at[pl.ds(base, CHUNK)])


def kernel(node_species, E0):
    mesh = plsc.VectorSubcoreMesh(core_axis_name="c", subcore_axis_name="s")
    run = pl.kernel(
        _gather_body,
        out_type=jax.ShapeDtypeStruct((N_NODES,), jnp.float32),
        mesh=mesh,
        scratch_types=[
            pltpu.VMEM((NUM_SPECIES,), jnp.float32),
            pltpu.VMEM((CHUNK,), jnp.int32),
            pltpu.VMEM((CHUNK,), jnp.float32),
            pltpu.SemaphoreType.DMA,
            pltpu.SemaphoreType.DMA,
        ],
        compiler_params=pltpu.CompilerParams(
            needs_layout_passes=False,
            disable_bounds_checks=True,
            disable_semaphore_checks=True,
        ),
    )
    return run(node_species.astype(jnp.int32), E0.astype(jnp.float32))
